# baseline (speedup 1.0000x reference)
import jax
import jax.numpy as jnp
from jax.experimental import pallas as pl
from jax.experimental.pallas import tpu as pltpu

B = 16
N = 4096
D = 64
TEMPERATURE = 0.05

CHUNK = 1024
NCHUNKS = (B * N) // CHUNK
CHUNKS_PER_ROW = N // CHUNK


def _body(len_ref, o_ref, s_ref, w_ref, r_ref, wl_out, w_out):
    pid = pl.program_id(0)
    b = pid // CHUNKS_PER_ROW
    n_base = (pid % CHUNKS_PER_ROW) * CHUNK

    o = o_ref[...]
    s = s_ref[...]
    logits = jnp.sum(o * s, axis=1) * (1.0 / TEMPERATURE)

    n_idx = n_base + jax.lax.broadcasted_iota(jnp.int32, (CHUNK,), 0)
    valid = (n_idx < len_ref[b]).astype(jnp.float32)
    w = w_ref[...] * valid
    t = r_ref[...]

    bce = jnp.maximum(logits, 0.0) - logits * t + jnp.log1p(jnp.exp(-jnp.abs(logits)))

    @pl.when(pid == 0)
    def _init():
        wl_out[0, 0] = 0.0
        w_out[0, 0] = 0.0

    wl_out[0, 0] += jnp.sum(bce * w)
    w_out[0, 0] += jnp.sum(w)


def kernel(lengths, output_embeddings, supervision_ids, supervision_embeddings, supervision_weights, supervision_ratings):
    del supervision_ids
    o2 = output_embeddings.reshape(B * N, D)
    s2 = supervision_embeddings.reshape(B * N, D)
    w2 = supervision_weights.reshape(B * N)
    r2 = supervision_ratings.reshape(B * N)

    wl, wsum = pl.pallas_call(
        _body,
        grid=(NCHUNKS,),
        in_specs=[
            pl.BlockSpec(memory_space=pltpu.SMEM),
            pl.BlockSpec((CHUNK, D), lambda i: (i, 0)),
            pl.BlockSpec((CHUNK, D), lambda i: (i, 0)),
            pl.BlockSpec((CHUNK,), lambda i: (i,)),
            pl.BlockSpec((CHUNK,), lambda i: (i,)),
        ],
        out_specs=[
            pl.BlockSpec(memory_space=pltpu.SMEM),
            pl.BlockSpec(memory_space=pltpu.SMEM),
        ],
        out_shape=[
            jax.ShapeDtypeStruct((1, 1), jnp.float32),
            jax.ShapeDtypeStruct((1, 1), jnp.float32),
        ],
    )(lengths, o2, s2, w2, r2)
    return (wl[0, 0] / wsum[0, 0]).astype(jnp.float32)
